# baseline (speedup 1.0000x reference)
import jax
import jax.numpy as jnp
from jax.experimental import pallas as pl
from jax.experimental.pallas import tpu as pltpu

_N = 256
_S = 4


def _fused_kernel(locs_ref, shot_ref, Wg_ref, vs_ref, Ad_ref, bg_ref,
                  Wt0_ref, Wt1_ref, Wt2_ref, bt_ref,
                  W1_ref, b1_ref, W2_ref, b2_ref,
                  shot_out_ref, locs_out_ref, rlocs_ref, rshot_ref):
    f32 = jnp.float32
    row = jax.lax.broadcasted_iota(jnp.int32, (_N, _N), 0)
    col = jax.lax.broadcasted_iota(jnp.int32, (_N, _N), 1)
    ridx = jax.lax.broadcasted_iota(jnp.int32, (_N, 16), 0)

    Wg = Wg_ref[...]
    for i in range(_S):
        x = locs_ref[i]
        h = x[:, 0:1] * Wg[0:1, :] + x[:, 1:2] * Wg[1:2, :]

        e_d = jnp.dot(h, Ad_ref[...], preferred_element_type=f32)
        a_s_row = jax.lax.dot_general(
            vs_ref[...], h, (((1,), (1,)), ((), ())),
            preferred_element_type=f32)
        e = e_d + a_s_row
        e = jnp.where(e >= 0, e, 0.2 * e)
        e = jnp.where(row == col, f32(-1e30), e)
        m = jnp.max(e, axis=1, keepdims=True)
        p = jnp.exp(e - m)
        alpha = p / jnp.sum(p, axis=1, keepdims=True)
        gat = jnp.dot(alpha, h, preferred_element_type=f32)
        gat = jnp.maximum(gat + bg_ref[...], 0.0)

        s0 = shot_ref[i]
        s1 = jnp.where(ridx >= 1, pltpu.roll(s0, 1, 0), 0.0)
        s2 = jnp.where(ridx >= 2, pltpu.roll(s0, 2, 0), 0.0)
        y = (jnp.dot(s0, Wt2_ref[...], preferred_element_type=f32)
             + jnp.dot(s1, Wt1_ref[...], preferred_element_type=f32)
             + jnp.dot(s2, Wt0_ref[...], preferred_element_type=f32))
        tcn = jnp.maximum(y + bt_ref[...], 0.0)

        z = (jnp.dot(gat, W1_ref[0:16, :], preferred_element_type=f32)
             + jnp.dot(tcn, W1_ref[16:32, :], preferred_element_type=f32)
             + b1_ref[...])
        z = jnp.where(z >= 0, z, 0.01 * z)
        o = jnp.dot(z, W2_ref[...], preferred_element_type=f32) + b2_ref[...]

        shot_out_ref[i] = o[:, 0:1]
        locs_out_ref[i] = o[:, 1:2]
        rlocs_ref[i] = o[:, 2:4]
        rshot_ref[i] = o[:, 4:20]


def kernel(locs, shot, W_gat, att_src, att_dst, b_gat, W_tcn, b_tcn,
           W_s1, b_s1, W_s2, b_s2, W_l1, b_l1, W_l2, b_l2,
           W_rl, b_rl, W_rs, b_rs):
    B, N, _ = locs.shape
    f32 = jnp.float32

    row = lambda v: v.reshape(1, -1).astype(f32)
    Wt = jnp.transpose(W_tcn, (1, 0, 2))
    W1 = jnp.concatenate([W_s1, W_l1], axis=1)
    b1 = jnp.concatenate([b_s1, b_l1]).reshape(1, 32)
    z16 = jnp.zeros((16, 1), f32)
    W2s = jnp.concatenate([W_s2, z16], axis=0)
    W2l = jnp.concatenate([z16, W_l2], axis=0)
    W2 = jnp.concatenate([W2s, W2l, W2l @ W_rl, W2s @ W_rs], axis=1)
    b2 = jnp.concatenate([
        b_s2, b_l2,
        b_l2 @ W_rl + b_rl, b_s2 @ W_rs + b_rs,
    ]).reshape(1, 20)
    args = (
        locs, shot, W_gat,
        row(att_src),
        (att_dst.astype(f32)[:, None] * jnp.ones((1, N), f32)),
        row(b_gat),
        Wt[:, :, 0], Wt[:, :, 1], Wt[:, :, 2], row(b_tcn),
        W1, b1, W2, b2,
    )

    batch3 = lambda d: pl.BlockSpec((_S, N, d), lambda b: (b, 0, 0))
    full2 = lambda a: pl.BlockSpec(a.shape, lambda b: (0,) * a.ndim)
    in_specs = [batch3(2), batch3(16)] + [full2(a) for a in args[2:]]

    out_shape = (
        jax.ShapeDtypeStruct((B, N, 1), f32),
        jax.ShapeDtypeStruct((B, N, 1), f32),
        jax.ShapeDtypeStruct((B, N, 2), f32),
        jax.ShapeDtypeStruct((B, N, 16), f32),
    )
    out_specs = (batch3(1), batch3(1), batch3(2), batch3(16))

    return pl.pallas_call(
        _fused_kernel,
        grid=(B // _S,),
        in_specs=in_specs,
        out_specs=out_specs,
        out_shape=out_shape,
        compiler_params=pltpu.CompilerParams(
            dimension_semantics=("parallel",),
        ),
    )(*args)


# speedup vs baseline: 1.0241x; 1.0241x over previous
import jax
import jax.numpy as jnp
from jax.experimental import pallas as pl
from jax.experimental.pallas import tpu as pltpu

_N = 256
_S = 8


def _fused_kernel(locs_ref, shot_ref, Wg_ref, vs_ref, Ad_ref, bg_ref,
                  Wt0_ref, Wt1_ref, Wt2_ref, bt_ref,
                  W1_ref, b1_ref, W2_ref, b2_ref,
                  Wrl_ref, brl_ref, Wrs_ref, brs_ref,
                  shot_out_ref, locs_out_ref, rlocs_ref, rshot_ref):
    f32 = jnp.float32
    row = jax.lax.broadcasted_iota(jnp.int32, (_N, _N), 0)
    col = jax.lax.broadcasted_iota(jnp.int32, (_N, _N), 1)
    ridx = jax.lax.broadcasted_iota(jnp.int32, (_N, 16), 0)

    Wg = Wg_ref[...]
    for i in range(_S):
        x = locs_ref[i]
        h = x[:, 0:1] * Wg[0:1, :] + x[:, 1:2] * Wg[1:2, :]

        e_d = jnp.dot(h, Ad_ref[...], preferred_element_type=f32)
        a_s_row = jax.lax.dot_general(
            vs_ref[...], h, (((1,), (1,)), ((), ())),
            preferred_element_type=f32)
        e = e_d + a_s_row
        e = jnp.where(e >= 0, e, 0.2 * e)
        e = jnp.where(row == col, f32(-1e30), e)
        m = jnp.max(e, axis=1, keepdims=True)
        p = jnp.exp(e - m)
        alpha = p / jnp.sum(p, axis=1, keepdims=True)
        gat = jnp.dot(alpha, h, preferred_element_type=f32)
        gat = jnp.maximum(gat + bg_ref[...], 0.0)

        s0 = shot_ref[i]
        s1 = jnp.where(ridx >= 1, pltpu.roll(s0, 1, 0), 0.0)
        s2 = jnp.where(ridx >= 2, pltpu.roll(s0, 2, 0), 0.0)
        y = (jnp.dot(s0, Wt2_ref[...], preferred_element_type=f32)
             + jnp.dot(s1, Wt1_ref[...], preferred_element_type=f32)
             + jnp.dot(s2, Wt0_ref[...], preferred_element_type=f32))
        tcn = jnp.maximum(y + bt_ref[...], 0.0)

        z = (jnp.dot(gat, W1_ref[0:16, :], preferred_element_type=f32)
             + jnp.dot(tcn, W1_ref[16:32, :], preferred_element_type=f32)
             + b1_ref[...])
        z = jnp.where(z >= 0, z, 0.01 * z)
        o2 = jnp.dot(z, W2_ref[...], preferred_element_type=f32) + b2_ref[...]
        so = o2[:, 0:1]
        lo = o2[:, 1:2]

        shot_out_ref[i] = so
        locs_out_ref[i] = lo
        rlocs_ref[i] = lo * Wrl_ref[...] + brl_ref[...]
        rshot_ref[i] = so * Wrs_ref[...] + brs_ref[...]


def kernel(locs, shot, W_gat, att_src, att_dst, b_gat, W_tcn, b_tcn,
           W_s1, b_s1, W_s2, b_s2, W_l1, b_l1, W_l2, b_l2,
           W_rl, b_rl, W_rs, b_rs):
    B, N, _ = locs.shape
    f32 = jnp.float32

    row = lambda v: v.reshape(1, -1).astype(f32)
    Wt = jnp.transpose(W_tcn, (1, 0, 2))
    W1 = jnp.concatenate([W_s1, W_l1], axis=1)
    b1 = jnp.concatenate([b_s1, b_l1]).reshape(1, 32)
    z16 = jnp.zeros((16, 1), f32)
    W2 = jnp.concatenate([
        jnp.concatenate([W_s2, z16], axis=1),
        jnp.concatenate([z16, W_l2], axis=1),
    ], axis=0)
    b2 = jnp.concatenate([b_s2, b_l2]).reshape(1, 2)
    args = (
        locs, shot, W_gat,
        row(att_src),
        (att_dst.astype(f32)[:, None] * jnp.ones((1, N), f32)),
        row(b_gat),
        Wt[:, :, 0], Wt[:, :, 1], Wt[:, :, 2], row(b_tcn),
        W1, b1, W2, b2,
        W_rl, row(b_rl), W_rs, row(b_rs),
    )

    batch3 = lambda d: pl.BlockSpec((_S, N, d), lambda b: (b, 0, 0))
    full2 = lambda a: pl.BlockSpec(a.shape, lambda b: (0,) * a.ndim)
    in_specs = [batch3(2), batch3(16)] + [full2(a) for a in args[2:]]

    out_shape = (
        jax.ShapeDtypeStruct((B, N, 1), f32),
        jax.ShapeDtypeStruct((B, N, 1), f32),
        jax.ShapeDtypeStruct((B, N, 2), f32),
        jax.ShapeDtypeStruct((B, N, 16), f32),
    )
    out_specs = (batch3(1), batch3(1), batch3(2), batch3(16))

    return pl.pallas_call(
        _fused_kernel,
        grid=(B // _S,),
        in_specs=in_specs,
        out_specs=out_specs,
        out_shape=out_shape,
        compiler_params=pltpu.CompilerParams(
            dimension_semantics=("parallel",),
        ),
    )(*args)


# speedup vs baseline: 1.0275x; 1.0033x over previous
import jax
import jax.numpy as jnp
from jax.experimental import pallas as pl
from jax.experimental.pallas import tpu as pltpu

_N = 256
_S = 8


def _fused_kernel(locs_ref, shot_ref, Wg_ref, asrc_ref, adst_ref, bg_ref,
                  Wtcn_ref, bt_ref, Ws1_ref, bs1_ref, Ws2_ref, bs2_ref,
                  Wl1_ref, bl1_ref, Wl2_ref, bl2_ref,
                  Wrl_ref, brl_ref, Wrs_ref, brs_ref,
                  shot_out_ref, locs_out_ref, rlocs_ref, rshot_ref):
    f32 = jnp.float32
    row = jax.lax.broadcasted_iota(jnp.int32, (_N, _N), 0)
    col = jax.lax.broadcasted_iota(jnp.int32, (_N, _N), 1)
    ridx = jax.lax.broadcasted_iota(jnp.int32, (_N, 16), 0)
    Wg = Wg_ref[...]
    adst_col = jnp.transpose(adst_ref[...])
    Ad = jnp.dot(adst_col, jnp.ones((1, _N), f32),
                 preferred_element_type=f32)
    Wt0 = jnp.transpose(Wtcn_ref[:, :, 0])
    Wt1 = jnp.transpose(Wtcn_ref[:, :, 1])
    Wt2 = jnp.transpose(Wtcn_ref[:, :, 2])

    for i in range(_S):
        x = locs_ref[i]
        h = x[:, 0:1] * Wg[0:1, :] + x[:, 1:2] * Wg[1:2, :]

        e_d = jnp.dot(h, Ad, preferred_element_type=f32)
        a_s_row = jax.lax.dot_general(
            asrc_ref[...], h, (((1,), (1,)), ((), ())),
            preferred_element_type=f32)
        e = e_d + a_s_row
        e = jnp.where(e >= 0, e, 0.2 * e)
        e = jnp.where(row == col, f32(-1e30), e)
        m = jnp.max(e, axis=1, keepdims=True)
        p = jnp.exp(e - m)
        alpha = p / jnp.sum(p, axis=1, keepdims=True)
        gat = jnp.dot(alpha, h, preferred_element_type=f32)
        gat = jnp.maximum(gat + bg_ref[...], 0.0)

        s0 = shot_ref[i]
        s1 = jnp.where(ridx >= 1, pltpu.roll(s0, 1, 0), 0.0)
        s2 = jnp.where(ridx >= 2, pltpu.roll(s0, 2, 0), 0.0)
        y = (jnp.dot(s0, Wt2, preferred_element_type=f32)
             + jnp.dot(s1, Wt1, preferred_element_type=f32)
             + jnp.dot(s2, Wt0, preferred_element_type=f32))
        tcn = jnp.maximum(y + bt_ref[...], 0.0)

        zs = (jnp.dot(gat, Ws1_ref[0:16, :], preferred_element_type=f32)
              + jnp.dot(tcn, Ws1_ref[16:32, :], preferred_element_type=f32)
              + bs1_ref[...])
        zs = jnp.where(zs >= 0, zs, 0.01 * zs)
        so = jnp.dot(zs, Ws2_ref[...], preferred_element_type=f32) + bs2_ref[...]

        zl = (jnp.dot(gat, Wl1_ref[0:16, :], preferred_element_type=f32)
              + jnp.dot(tcn, Wl1_ref[16:32, :], preferred_element_type=f32)
              + bl1_ref[...])
        zl = jnp.where(zl >= 0, zl, 0.01 * zl)
        lo = jnp.dot(zl, Wl2_ref[...], preferred_element_type=f32) + bl2_ref[...]

        shot_out_ref[i] = so
        locs_out_ref[i] = lo
        rlocs_ref[i] = lo * Wrl_ref[...] + brl_ref[...]
        rshot_ref[i] = so * Wrs_ref[...] + brs_ref[...]


def kernel(locs, shot, W_gat, att_src, att_dst, b_gat, W_tcn, b_tcn,
           W_s1, b_s1, W_s2, b_s2, W_l1, b_l1, W_l2, b_l2,
           W_rl, b_rl, W_rs, b_rs):
    B, N, _ = locs.shape
    f32 = jnp.float32

    row = lambda v: v.reshape(1, -1)
    args = (
        locs, shot, W_gat,
        row(att_src), row(att_dst), row(b_gat),
        W_tcn, row(b_tcn),
        W_s1, row(b_s1), W_s2, row(b_s2),
        W_l1, row(b_l1), W_l2, row(b_l2),
        W_rl, row(b_rl), W_rs, row(b_rs),
    )

    batch3 = lambda d: pl.BlockSpec((_S, N, d), lambda b: (b, 0, 0))
    full = lambda a: pl.BlockSpec(a.shape, lambda b: (0,) * a.ndim)
    in_specs = [batch3(2), batch3(16)] + [full(a) for a in args[2:]]

    out_shape = (
        jax.ShapeDtypeStruct((B, N, 1), f32),
        jax.ShapeDtypeStruct((B, N, 1), f32),
        jax.ShapeDtypeStruct((B, N, 2), f32),
        jax.ShapeDtypeStruct((B, N, 16), f32),
    )
    out_specs = (batch3(1), batch3(1), batch3(2), batch3(16))

    return pl.pallas_call(
        _fused_kernel,
        grid=(B // _S,),
        in_specs=in_specs,
        out_specs=out_specs,
        out_shape=out_shape,
        compiler_params=pltpu.CompilerParams(
            dimension_semantics=("parallel",),
        ),
    )(*args)


# speedup vs baseline: 1.8671x; 1.8172x over previous
import jax
import jax.numpy as jnp
from jax.experimental import pallas as pl
from jax.experimental.pallas import tpu as pltpu

_N = 256
_S = 8


def _fused_kernel(locs_ref, shot_ref, WgT_ref, asrc_ref, adst_ref, bg_ref,
                  Wtcn_ref, bt_ref, Ws1T_ref, bs1_ref, Ws2T_ref, bs2_ref,
                  Wl1T_ref, bl1_ref, Wl2T_ref, bl2_ref,
                  Wrl_ref, brl_ref, Wrs_ref, brs_ref,
                  so_ref, lo_ref, rlocs_ref, rshot_ref):
    f32 = jnp.float32
    srow = jax.lax.broadcasted_iota(jnp.int32, (_N, _N), 0)
    dcol = jax.lax.broadcasted_iota(jnp.int32, (_N, _N), 1)
    cidx = jax.lax.broadcasted_iota(jnp.int32, (16, _N), 1)
    ones_row = jnp.ones((1, _N), f32)
    rep = lambda r: jnp.dot(jnp.transpose(r), ones_row,
                            preferred_element_type=f32)
    Asrc = rep(asrc_ref[...])
    Bg = rep(bg_ref[...])
    Bt = rep(bt_ref[...])
    Bs1 = rep(bs1_ref[...])
    Bl1 = rep(bl1_ref[...])
    Rrs = rep(Wrs_ref[...])
    Brs = rep(brs_ref[...])
    Rrl = rep(Wrl_ref[...])
    Brl = rep(brl_ref[...])
    WgT = WgT_ref[...]
    Wt0 = Wtcn_ref[0]
    Wt1 = Wtcn_ref[1]
    Wt2 = Wtcn_ref[2]

    for i in range(_S):
        xT = locs_ref[i]
        hT = jnp.dot(WgT, xT, preferred_element_type=f32)

        a_d_row = jnp.dot(adst_ref[...], hT, preferred_element_type=f32)
        e_s = jax.lax.dot_general(hT, Asrc, (((0,), (0,)), ((), ())),
                                  preferred_element_type=f32)
        e = e_s + a_d_row
        e = jnp.where(e >= 0, e, 0.2 * e)
        e = jnp.where(srow == dcol, f32(-1e30), e)
        m = jnp.max(e, axis=0, keepdims=True)
        p = jnp.exp(e - m)
        ssum = jnp.sum(p, axis=0, keepdims=True)
        gat = jnp.dot(hT, p, preferred_element_type=f32) / ssum
        gat = jnp.maximum(gat + Bg, 0.0)

        s0 = shot_ref[i]
        s1 = jnp.where(cidx >= 1, pltpu.roll(s0, 1, 1), 0.0)
        s2 = jnp.where(cidx >= 2, pltpu.roll(s0, 2, 1), 0.0)
        y = (jnp.dot(Wt2, s0, preferred_element_type=f32)
             + jnp.dot(Wt1, s1, preferred_element_type=f32)
             + jnp.dot(Wt0, s2, preferred_element_type=f32))
        tcn = jnp.maximum(y + Bt, 0.0)

        zs = (jnp.dot(Ws1T_ref[:, 0:16], gat, preferred_element_type=f32)
              + jnp.dot(Ws1T_ref[:, 16:32], tcn, preferred_element_type=f32)
              + Bs1)
        zs = jnp.where(zs >= 0, zs, 0.01 * zs)
        so = jnp.dot(Ws2T_ref[...], zs, preferred_element_type=f32) + bs2_ref[...]

        zl = (jnp.dot(Wl1T_ref[:, 0:16], gat, preferred_element_type=f32)
              + jnp.dot(Wl1T_ref[:, 16:32], tcn, preferred_element_type=f32)
              + Bl1)
        zl = jnp.where(zl >= 0, zl, 0.01 * zl)
        lo = jnp.dot(Wl2T_ref[...], zl, preferred_element_type=f32) + bl2_ref[...]

        so_ref[i] = so
        lo_ref[i] = lo
        rlocs_ref[i] = lo * Rrl + Brl
        rshot_ref[i] = so * Rrs + Brs


def kernel(locs, shot, W_gat, att_src, att_dst, b_gat, W_tcn, b_tcn,
           W_s1, b_s1, W_s2, b_s2, W_l1, b_l1, W_l2, b_l2,
           W_rl, b_rl, W_rs, b_rs):
    B, N, _ = locs.shape
    f32 = jnp.float32

    tr = lambda a: jnp.transpose(a, (0, 2, 1))
    row = lambda v: v.reshape(1, -1)
    args = (
        tr(locs), tr(shot), W_gat.T,
        row(att_src), row(att_dst), row(b_gat),
        jnp.transpose(W_tcn, (2, 0, 1)), row(b_tcn),
        W_s1.T, row(b_s1), W_s2.T, row(b_s2),
        W_l1.T, row(b_l1), W_l2.T, row(b_l2),
        W_rl, row(b_rl), W_rs, row(b_rs),
    )

    batch3 = lambda d: pl.BlockSpec((_S, d, N), lambda b: (b, 0, 0))
    full = lambda a: pl.BlockSpec(a.shape, lambda b: (0,) * a.ndim)
    in_specs = [batch3(2), batch3(16)] + [full(a) for a in args[2:]]

    out_shape = (
        jax.ShapeDtypeStruct((B, 1, N), f32),
        jax.ShapeDtypeStruct((B, 1, N), f32),
        jax.ShapeDtypeStruct((B, 2, N), f32),
        jax.ShapeDtypeStruct((B, 16, N), f32),
    )
    out_specs = (batch3(1), batch3(1), batch3(2), batch3(16))

    outs = pl.pallas_call(
        _fused_kernel,
        grid=(B // _S,),
        in_specs=in_specs,
        out_specs=out_specs,
        out_shape=out_shape,
        compiler_params=pltpu.CompilerParams(
            dimension_semantics=("parallel",),
        ),
    )(*args)
    return tuple(tr(o) for o in outs)


# speedup vs baseline: 4.5706x; 2.4480x over previous
import jax
import jax.numpy as jnp
from jax.experimental import pallas as pl
from jax.experimental.pallas import tpu as pltpu

_N = 256
_S = 8


def _fused_kernel(locs_ref, shot_ref, WgT_ref, asrc_ref, adst_ref, bg_ref,
                  Wtcn_ref, bt_ref, Ws1T_ref, bs1_ref, Ws2T_ref, bs2_ref,
                  Wl1T_ref, bl1_ref, Wl2T_ref, bl2_ref,
                  Wrl_ref, brl_ref, Wrs_ref, brs_ref,
                  so_ref, lo_ref, rlocs_ref, rshot_ref):
    f32 = jnp.float32
    srow = jax.lax.broadcasted_iota(jnp.int32, (_N, _N), 0)
    dcol = jax.lax.broadcasted_iota(jnp.int32, (_N, _N), 1)
    cidx = jax.lax.broadcasted_iota(jnp.int32, (16, _N), 1)
    ones_row = jnp.ones((1, _N), f32)
    rep = lambda r: jnp.dot(jnp.transpose(r), ones_row,
                            preferred_element_type=f32)
    Asrc = rep(asrc_ref[...])
    Bg = rep(bg_ref[...])
    Bt = rep(bt_ref[...])
    Bs1 = rep(bs1_ref[...])
    Bl1 = rep(bl1_ref[...])
    Rrs = rep(Wrs_ref[...])
    Brs = rep(brs_ref[...])
    Rrl = rep(Wrl_ref[...])
    Brl = rep(brl_ref[...])
    WgT = WgT_ref[...]
    Wt0 = Wtcn_ref[0]
    Wt1 = Wtcn_ref[1]
    Wt2 = Wtcn_ref[2]

    R = range(_S)
    dot = lambda a, b: jnp.dot(a, b, preferred_element_type=f32)

    hs = [dot(WgT, locs_ref[i]) for i in R]
    ads = [dot(adst_ref[...], hs[i]) for i in R]
    ess = [jax.lax.dot_general(hs[i], Asrc, (((0,), (0,)), ((), ())),
                               preferred_element_type=f32) for i in R]
    es = [jnp.where(srow == dcol, f32(-1e30),
                    jnp.where(e >= 0, e, 0.2 * e))
          for e in (ess[i] + ads[i] for i in R)]
    ms = [jnp.max(es[i], axis=0, keepdims=True) for i in R]
    ps = [jnp.exp(es[i] - ms[i]) for i in R]
    ssums = [jnp.sum(ps[i], axis=0, keepdims=True) for i in R]
    gats = [jnp.maximum(dot(hs[i], ps[i]) / ssums[i] + Bg, 0.0) for i in R]

    s0s = [shot_ref[i] for i in R]
    s1s = [jnp.where(cidx >= 1, pltpu.roll(s, 1, 1), 0.0) for s in s0s]
    s2s = [jnp.where(cidx >= 2, pltpu.roll(s, 2, 1), 0.0) for s in s0s]
    tcns = [jnp.maximum(dot(Wt2, s0s[i]) + dot(Wt1, s1s[i])
                        + dot(Wt0, s2s[i]) + Bt, 0.0) for i in R]

    leaky = lambda z: jnp.where(z >= 0, z, 0.01 * z)
    zss = [leaky(dot(Ws1T_ref[:, 0:16], gats[i])
                 + dot(Ws1T_ref[:, 16:32], tcns[i]) + Bs1) for i in R]
    sos = [dot(Ws2T_ref[...], zss[i]) + bs2_ref[...] for i in R]
    zls = [leaky(dot(Wl1T_ref[:, 0:16], gats[i])
                 + dot(Wl1T_ref[:, 16:32], tcns[i]) + Bl1) for i in R]
    los = [dot(Wl2T_ref[...], zls[i]) + bl2_ref[...] for i in R]

    for i in R:
        so_ref[i] = sos[i]
        lo_ref[i] = los[i]
        rlocs_ref[i] = los[i] * Rrl + Brl
        rshot_ref[i] = sos[i] * Rrs + Brs


def kernel(locs, shot, W_gat, att_src, att_dst, b_gat, W_tcn, b_tcn,
           W_s1, b_s1, W_s2, b_s2, W_l1, b_l1, W_l2, b_l2,
           W_rl, b_rl, W_rs, b_rs):
    B, N, _ = locs.shape
    f32 = jnp.float32

    tr = lambda a: jnp.transpose(a, (0, 2, 1))
    row = lambda v: v.reshape(1, -1)
    args = (
        tr(locs), tr(shot), W_gat.T,
        row(att_src), row(att_dst), row(b_gat),
        jnp.transpose(W_tcn, (2, 0, 1)), row(b_tcn),
        W_s1.T, row(b_s1), W_s2.T, row(b_s2),
        W_l1.T, row(b_l1), W_l2.T, row(b_l2),
        W_rl, row(b_rl), W_rs, row(b_rs),
    )

    batch3 = lambda d: pl.BlockSpec((_S, d, N), lambda b: (b, 0, 0))
    full = lambda a: pl.BlockSpec(a.shape, lambda b: (0,) * a.ndim)
    in_specs = [batch3(2), batch3(16)] + [full(a) for a in args[2:]]

    out_shape = (
        jax.ShapeDtypeStruct((B, 1, N), f32),
        jax.ShapeDtypeStruct((B, 1, N), f32),
        jax.ShapeDtypeStruct((B, 2, N), f32),
        jax.ShapeDtypeStruct((B, 16, N), f32),
    )
    out_specs = (batch3(1), batch3(1), batch3(2), batch3(16))

    outs = pl.pallas_call(
        _fused_kernel,
        grid=(B // _S,),
        in_specs=in_specs,
        out_specs=out_specs,
        out_shape=out_shape,
        compiler_params=pltpu.CompilerParams(
            dimension_semantics=("parallel",),
        ),
    )(*args)
    return tuple(tr(o) for o in outs)


# speedup vs baseline: 5.4820x; 1.1994x over previous
import jax
import jax.numpy as jnp
from jax.experimental import pallas as pl
from jax.experimental.pallas import tpu as pltpu

_N = 256
_S = 16


def _fused_kernel(locs_ref, shot_ref, WgT_ref, asrc_ref, adst_ref, bg_ref,
                  Wtcn_ref, bt_ref, Ws1T_ref, bs1_ref, Ws2T_ref, bs2_ref,
                  Wl1T_ref, bl1_ref, Wl2T_ref, bl2_ref,
                  Wrl_ref, brl_ref, Wrs_ref, brs_ref,
                  so_ref, lo_ref, rlocs_ref, rshot_ref):
    f32 = jnp.float32
    srow = jax.lax.broadcasted_iota(jnp.int32, (_N, _N), 0)
    dcol = jax.lax.broadcasted_iota(jnp.int32, (_N, _N), 1)
    cidx = jax.lax.broadcasted_iota(jnp.int32, (16, _N), 1)
    ones_row = jnp.ones((1, _N), f32)
    rep = lambda r: jnp.dot(jnp.transpose(r), ones_row,
                            preferred_element_type=f32)
    Asrc = rep(asrc_ref[...])
    Bg = rep(bg_ref[...])
    Bt = rep(bt_ref[...])
    Bs1 = rep(bs1_ref[...])
    Bl1 = rep(bl1_ref[...])
    Rrs = rep(Wrs_ref[...])
    Brs = rep(brs_ref[...])
    Rrl = rep(Wrl_ref[...])
    Brl = rep(brl_ref[...])
    WgT = WgT_ref[...]
    Wt0 = Wtcn_ref[0]
    Wt1 = Wtcn_ref[1]
    Wt2 = Wtcn_ref[2]

    R = range(_S)
    dot = lambda a, b: jnp.dot(a, b, preferred_element_type=f32)

    hs = [dot(WgT, locs_ref[i]) for i in R]
    ads = [dot(adst_ref[...], hs[i]) for i in R]
    ess = [jax.lax.dot_general(hs[i], Asrc, (((0,), (0,)), ((), ())),
                               preferred_element_type=f32) for i in R]
    es = [jnp.where(srow == dcol, f32(-1e30),
                    jnp.where(e >= 0, e, 0.2 * e))
          for e in (ess[i] + ads[i] for i in R)]
    ms = [jnp.max(es[i], axis=0, keepdims=True) for i in R]
    ps = [jnp.exp(es[i] - ms[i]) for i in R]
    ssums = [jnp.sum(ps[i], axis=0, keepdims=True) for i in R]
    gats = [jnp.maximum(dot(hs[i], ps[i]) / ssums[i] + Bg, 0.0) for i in R]

    s0s = [shot_ref[i] for i in R]
    s1s = [jnp.where(cidx >= 1, pltpu.roll(s, 1, 1), 0.0) for s in s0s]
    s2s = [jnp.where(cidx >= 2, pltpu.roll(s, 2, 1), 0.0) for s in s0s]
    tcns = [jnp.maximum(dot(Wt2, s0s[i]) + dot(Wt1, s1s[i])
                        + dot(Wt0, s2s[i]) + Bt, 0.0) for i in R]

    leaky = lambda z: jnp.where(z >= 0, z, 0.01 * z)
    zss = [leaky(dot(Ws1T_ref[:, 0:16], gats[i])
                 + dot(Ws1T_ref[:, 16:32], tcns[i]) + Bs1) for i in R]
    sos = [dot(Ws2T_ref[...], zss[i]) + bs2_ref[...] for i in R]
    zls = [leaky(dot(Wl1T_ref[:, 0:16], gats[i])
                 + dot(Wl1T_ref[:, 16:32], tcns[i]) + Bl1) for i in R]
    los = [dot(Wl2T_ref[...], zls[i]) + bl2_ref[...] for i in R]

    for i in R:
        so_ref[i] = sos[i]
        lo_ref[i] = los[i]
        rlocs_ref[i] = los[i] * Rrl + Brl
        rshot_ref[i] = sos[i] * Rrs + Brs


def kernel(locs, shot, W_gat, att_src, att_dst, b_gat, W_tcn, b_tcn,
           W_s1, b_s1, W_s2, b_s2, W_l1, b_l1, W_l2, b_l2,
           W_rl, b_rl, W_rs, b_rs):
    B, N, _ = locs.shape
    f32 = jnp.float32

    tr = lambda a: jnp.transpose(a, (0, 2, 1))
    row = lambda v: v.reshape(1, -1)
    args = (
        tr(locs), tr(shot), W_gat.T,
        row(att_src), row(att_dst), row(b_gat),
        jnp.transpose(W_tcn, (2, 0, 1)), row(b_tcn),
        W_s1.T, row(b_s1), W_s2.T, row(b_s2),
        W_l1.T, row(b_l1), W_l2.T, row(b_l2),
        W_rl, row(b_rl), W_rs, row(b_rs),
    )

    batch3 = lambda d: pl.BlockSpec((_S, d, N), lambda b: (b, 0, 0))
    full = lambda a: pl.BlockSpec(a.shape, lambda b: (0,) * a.ndim)
    in_specs = [batch3(2), batch3(16)] + [full(a) for a in args[2:]]

    out_shape = (
        jax.ShapeDtypeStruct((B, 1, N), f32),
        jax.ShapeDtypeStruct((B, 1, N), f32),
        jax.ShapeDtypeStruct((B, 2, N), f32),
        jax.ShapeDtypeStruct((B, 16, N), f32),
    )
    out_specs = (batch3(1), batch3(1), batch3(2), batch3(16))

    outs = pl.pallas_call(
        _fused_kernel,
        grid=(B // _S,),
        in_specs=in_specs,
        out_specs=out_specs,
        out_shape=out_shape,
        compiler_params=pltpu.CompilerParams(
            dimension_semantics=("parallel",),
        ),
    )(*args)
    return tuple(tr(o) for o in outs)


# speedup vs baseline: 6.2392x; 1.1381x over previous
import jax
import jax.numpy as jnp
from jax.experimental import pallas as pl
from jax.experimental.pallas import tpu as pltpu

_N = 256
_S = 32


def _fused_kernel(locs_ref, shot_ref, WgT_ref, asrc_ref, adst_ref, bg_ref,
                  Wtcn_ref, bt_ref, Ws1T_ref, bs1_ref, Ws2T_ref, bs2_ref,
                  Wl1T_ref, bl1_ref, Wl2T_ref, bl2_ref,
                  Wrl_ref, brl_ref, Wrs_ref, brs_ref,
                  so_ref, lo_ref, rlocs_ref, rshot_ref):
    f32 = jnp.float32
    srow = jax.lax.broadcasted_iota(jnp.int32, (_N, _N), 0)
    dcol = jax.lax.broadcasted_iota(jnp.int32, (_N, _N), 1)
    cidx = jax.lax.broadcasted_iota(jnp.int32, (16, _N), 1)
    ones_row = jnp.ones((1, _N), f32)
    rep = lambda r: jnp.dot(jnp.transpose(r), ones_row,
                            preferred_element_type=f32)
    Asrc = rep(asrc_ref[...])
    Bg = rep(bg_ref[...])
    Bt = rep(bt_ref[...])
    Bs1 = rep(bs1_ref[...])
    Bl1 = rep(bl1_ref[...])
    Rrs = rep(Wrs_ref[...])
    Brs = rep(brs_ref[...])
    Rrl = rep(Wrl_ref[...])
    Brl = rep(brl_ref[...])
    WgT = WgT_ref[...]
    Wt0 = Wtcn_ref[0]
    Wt1 = Wtcn_ref[1]
    Wt2 = Wtcn_ref[2]

    R = range(_S)
    dot = lambda a, b: jnp.dot(a, b, preferred_element_type=f32)

    hs = [dot(WgT, locs_ref[i]) for i in R]
    ads = [dot(adst_ref[...], hs[i]) for i in R]
    ess = [jax.lax.dot_general(hs[i], Asrc, (((0,), (0,)), ((), ())),
                               preferred_element_type=f32) for i in R]
    es = [jnp.where(srow == dcol, f32(-1e30),
                    jnp.where(e >= 0, e, 0.2 * e))
          for e in (ess[i] + ads[i] for i in R)]
    ms = [jnp.max(es[i], axis=0, keepdims=True) for i in R]
    ps = [jnp.exp(es[i] - ms[i]) for i in R]
    ssums = [jnp.sum(ps[i], axis=0, keepdims=True) for i in R]
    gats = [jnp.maximum(dot(hs[i], ps[i]) / ssums[i] + Bg, 0.0) for i in R]

    s0s = [shot_ref[i] for i in R]
    s1s = [jnp.where(cidx >= 1, pltpu.roll(s, 1, 1), 0.0) for s in s0s]
    s2s = [jnp.where(cidx >= 2, pltpu.roll(s, 2, 1), 0.0) for s in s0s]
    tcns = [jnp.maximum(dot(Wt2, s0s[i]) + dot(Wt1, s1s[i])
                        + dot(Wt0, s2s[i]) + Bt, 0.0) for i in R]

    leaky = lambda z: jnp.where(z >= 0, z, 0.01 * z)
    zss = [leaky(dot(Ws1T_ref[:, 0:16], gats[i])
                 + dot(Ws1T_ref[:, 16:32], tcns[i]) + Bs1) for i in R]
    sos = [dot(Ws2T_ref[...], zss[i]) + bs2_ref[...] for i in R]
    zls = [leaky(dot(Wl1T_ref[:, 0:16], gats[i])
                 + dot(Wl1T_ref[:, 16:32], tcns[i]) + Bl1) for i in R]
    los = [dot(Wl2T_ref[...], zls[i]) + bl2_ref[...] for i in R]

    for i in R:
        so_ref[i] = sos[i]
        lo_ref[i] = los[i]
        rlocs_ref[i] = los[i] * Rrl + Brl
        rshot_ref[i] = sos[i] * Rrs + Brs


def kernel(locs, shot, W_gat, att_src, att_dst, b_gat, W_tcn, b_tcn,
           W_s1, b_s1, W_s2, b_s2, W_l1, b_l1, W_l2, b_l2,
           W_rl, b_rl, W_rs, b_rs):
    B, N, _ = locs.shape
    f32 = jnp.float32

    tr = lambda a: jnp.transpose(a, (0, 2, 1))
    row = lambda v: v.reshape(1, -1)
    args = (
        tr(locs), tr(shot), W_gat.T,
        row(att_src), row(att_dst), row(b_gat),
        jnp.transpose(W_tcn, (2, 0, 1)), row(b_tcn),
        W_s1.T, row(b_s1), W_s2.T, row(b_s2),
        W_l1.T, row(b_l1), W_l2.T, row(b_l2),
        W_rl, row(b_rl), W_rs, row(b_rs),
    )

    batch3 = lambda d: pl.BlockSpec((_S, d, N), lambda b: (b, 0, 0))
    full = lambda a: pl.BlockSpec(a.shape, lambda b: (0,) * a.ndim)
    in_specs = [batch3(2), batch3(16)] + [full(a) for a in args[2:]]

    out_shape = (
        jax.ShapeDtypeStruct((B, 1, N), f32),
        jax.ShapeDtypeStruct((B, 1, N), f32),
        jax.ShapeDtypeStruct((B, 2, N), f32),
        jax.ShapeDtypeStruct((B, 16, N), f32),
    )
    out_specs = (batch3(1), batch3(1), batch3(2), batch3(16))

    outs = pl.pallas_call(
        _fused_kernel,
        grid=(B // _S,),
        in_specs=in_specs,
        out_specs=out_specs,
        out_shape=out_shape,
        compiler_params=pltpu.CompilerParams(
            dimension_semantics=("parallel",),
        ),
    )(*args)
    return tuple(tr(o) for o in outs)


# speedup vs baseline: 6.4527x; 1.0342x over previous
import jax
import jax.numpy as jnp
from jax.experimental import pallas as pl
from jax.experimental.pallas import tpu as pltpu

_N = 256
_S = 64


def _fused_kernel(locs_ref, shot_ref, WgT_ref, asrc_ref, adst_ref, bg_ref,
                  Wtcn_ref, bt_ref, Ws1T_ref, bs1_ref, Ws2T_ref, bs2_ref,
                  Wl1T_ref, bl1_ref, Wl2T_ref, bl2_ref,
                  Wrl_ref, brl_ref, Wrs_ref, brs_ref,
                  so_ref, lo_ref, rlocs_ref, rshot_ref):
    f32 = jnp.float32
    srow = jax.lax.broadcasted_iota(jnp.int32, (_N, _N), 0)
    dcol = jax.lax.broadcasted_iota(jnp.int32, (_N, _N), 1)
    cidx = jax.lax.broadcasted_iota(jnp.int32, (16, _N), 1)
    ones_row = jnp.ones((1, _N), f32)
    rep = lambda r: jnp.dot(jnp.transpose(r), ones_row,
                            preferred_element_type=f32)
    Asrc = rep(asrc_ref[...])
    Bg = rep(bg_ref[...])
    Bt = rep(bt_ref[...])
    Bs1 = rep(bs1_ref[...])
    Bl1 = rep(bl1_ref[...])
    Rrs = rep(Wrs_ref[...])
    Brs = rep(brs_ref[...])
    Rrl = rep(Wrl_ref[...])
    Brl = rep(brl_ref[...])
    WgT = WgT_ref[...]
    Wt0 = Wtcn_ref[0]
    Wt1 = Wtcn_ref[1]
    Wt2 = Wtcn_ref[2]

    R = range(_S)
    dot = lambda a, b: jnp.dot(a, b, preferred_element_type=f32)

    hs = [dot(WgT, locs_ref[i]) for i in R]
    ads = [dot(adst_ref[...], hs[i]) for i in R]
    ess = [jax.lax.dot_general(hs[i], Asrc, (((0,), (0,)), ((), ())),
                               preferred_element_type=f32) for i in R]
    es = [jnp.where(srow == dcol, f32(-1e30),
                    jnp.where(e >= 0, e, 0.2 * e))
          for e in (ess[i] + ads[i] for i in R)]
    ms = [jnp.max(es[i], axis=0, keepdims=True) for i in R]
    ps = [jnp.exp(es[i] - ms[i]) for i in R]
    ssums = [jnp.sum(ps[i], axis=0, keepdims=True) for i in R]
    gats = [jnp.maximum(dot(hs[i], ps[i]) / ssums[i] + Bg, 0.0) for i in R]

    s0s = [shot_ref[i] for i in R]
    s1s = [jnp.where(cidx >= 1, pltpu.roll(s, 1, 1), 0.0) for s in s0s]
    s2s = [jnp.where(cidx >= 2, pltpu.roll(s, 2, 1), 0.0) for s in s0s]
    tcns = [jnp.maximum(dot(Wt2, s0s[i]) + dot(Wt1, s1s[i])
                        + dot(Wt0, s2s[i]) + Bt, 0.0) for i in R]

    leaky = lambda z: jnp.where(z >= 0, z, 0.01 * z)
    zss = [leaky(dot(Ws1T_ref[:, 0:16], gats[i])
                 + dot(Ws1T_ref[:, 16:32], tcns[i]) + Bs1) for i in R]
    sos = [dot(Ws2T_ref[...], zss[i]) + bs2_ref[...] for i in R]
    zls = [leaky(dot(Wl1T_ref[:, 0:16], gats[i])
                 + dot(Wl1T_ref[:, 16:32], tcns[i]) + Bl1) for i in R]
    los = [dot(Wl2T_ref[...], zls[i]) + bl2_ref[...] for i in R]

    for i in R:
        so_ref[i] = sos[i]
        lo_ref[i] = los[i]
        rlocs_ref[i] = los[i] * Rrl + Brl
        rshot_ref[i] = sos[i] * Rrs + Brs


def kernel(locs, shot, W_gat, att_src, att_dst, b_gat, W_tcn, b_tcn,
           W_s1, b_s1, W_s2, b_s2, W_l1, b_l1, W_l2, b_l2,
           W_rl, b_rl, W_rs, b_rs):
    B, N, _ = locs.shape
    f32 = jnp.float32

    tr = lambda a: jnp.transpose(a, (0, 2, 1))
    row = lambda v: v.reshape(1, -1)
    args = (
        tr(locs), tr(shot), W_gat.T,
        row(att_src), row(att_dst), row(b_gat),
        jnp.transpose(W_tcn, (2, 0, 1)), row(b_tcn),
        W_s1.T, row(b_s1), W_s2.T, row(b_s2),
        W_l1.T, row(b_l1), W_l2.T, row(b_l2),
        W_rl, row(b_rl), W_rs, row(b_rs),
    )

    batch3 = lambda d: pl.BlockSpec((_S, d, N), lambda b: (b, 0, 0))
    full = lambda a: pl.BlockSpec(a.shape, lambda b: (0,) * a.ndim)
    in_specs = [batch3(2), batch3(16)] + [full(a) for a in args[2:]]

    out_shape = (
        jax.ShapeDtypeStruct((B, 1, N), f32),
        jax.ShapeDtypeStruct((B, 1, N), f32),
        jax.ShapeDtypeStruct((B, 2, N), f32),
        jax.ShapeDtypeStruct((B, 16, N), f32),
    )
    out_specs = (batch3(1), batch3(1), batch3(2), batch3(16))

    outs = pl.pallas_call(
        _fused_kernel,
        grid=(B // _S,),
        in_specs=in_specs,
        out_specs=out_specs,
        out_shape=out_shape,
        compiler_params=pltpu.CompilerParams(
            dimension_semantics=("parallel",),
        ),
    )(*args)
    return tuple(tr(o) for o in outs)
